# Initial kernel scaffold; baseline (speedup 1.0000x reference)
import jax
import jax.numpy as jnp
from jax.experimental import pallas as pl


def kernel(q, k, v, self_indices, neighbor_indices, Wq, bq, Wk, bk, Wv, bv, Wo, bo, ln1_w, ln1_b, W1, b1, W2, b2, ln2_w, ln2_b):
    raise NotImplementedError("write your pallas kernel here")


# speedup vs baseline: 860.5738x; 860.5738x over previous
import functools

import jax
import jax.numpy as jnp
from jax import lax
from jax.experimental import pallas as pl
from jax.experimental.pallas import tpu as pltpu
from jax.experimental.pallas import tpu_sc as plsc

_N = 10000
_E = 320000
_D = 128

_NC = 2
_NS = 16
_NW = _NC * _NS
_EPW = _E // _NW
_CHUNK = 640
_NPAD = _NS * _CHUNK

_ROWS = 2000


def _sc_count_body(idx_hbm, out_hbm, idx_v, ones_v, zeros_v, shared):
    cid = lax.axis_index("c")
    sid = lax.axis_index("s")
    wid = sid * _NC + cid

    def fill_ones(i, _):
        ones_v[pl.ds(i * 16, 16)] = jnp.full((16,), 1.0, jnp.float32)
        return 0

    lax.fori_loop(0, _EPW // 16, fill_ones, 0)

    def fill_zeros(i, _):
        zeros_v[pl.ds(i * 16, 16)] = jnp.zeros((16,), jnp.float32)
        return 0

    lax.fori_loop(0, _CHUNK // 16, fill_zeros, 0)

    pltpu.sync_copy(zeros_v, shared.at[pl.ds(sid * _CHUNK, _CHUNK)])
    pltpu.sync_copy(idx_hbm.at[pl.ds(wid * _EPW, _EPW)], idx_v)
    plsc.subcore_barrier()
    pltpu.sync_copy(ones_v, shared.at[idx_v], add=True)
    plsc.subcore_barrier()
    pltpu.sync_copy(
        shared.at[pl.ds(sid * _CHUNK, _CHUNK)],
        out_hbm.at[pl.ds(cid * _NPAD + sid * _CHUNK, _CHUNK)],
    )


@functools.cache
def _sc_count():
    return pl.kernel(
        _sc_count_body,
        out_type=jax.ShapeDtypeStruct((_NC * _NPAD,), jnp.float32),
        mesh=plsc.VectorSubcoreMesh(core_axis_name="c", subcore_axis_name="s",
                                    num_cores=_NC, num_subcores=_NS),
        scratch_types=[
            pltpu.VMEM((_EPW,), jnp.int32),
            pltpu.VMEM((_EPW,), jnp.float32),
            pltpu.VMEM((_CHUNK,), jnp.float32),
            pltpu.VMEM_SHARED((_NPAD,), jnp.float32),
        ],
    )


def _layernorm(x, w, b):
    m = jnp.mean(x, axis=-1, keepdims=True)
    var = jnp.mean((x - m) * (x - m), axis=-1, keepdims=True)
    return (x - m) * lax.rsqrt(var + 1e-5) * w + b


def _tc_body(v_ref, c0_ref, c1_ref, wv_ref, bv_ref, wo_ref, bo_ref,
             ln1w_ref, ln1b_ref, w1_ref, b1_ref, w2_ref, b2_ref,
             ln2w_ref, ln2b_ref, out_ref):
    xv = v_ref[:]
    ind = jnp.where(c0_ref[:] + c1_ref[:] > 0.0, 1.0, 0.0)
    vl = jnp.dot(xv, wv_ref[:], preferred_element_type=jnp.float32) + bv_ref[:]
    agg = vl * ind
    v2 = jnp.dot(agg, wo_ref[:], preferred_element_type=jnp.float32) + bo_ref[:]
    x = _layernorm(xv + v2, ln1w_ref[:], ln1b_ref[:])
    h = jnp.maximum(
        jnp.dot(x, w1_ref[:], preferred_element_type=jnp.float32) + b1_ref[:], 0.0)
    v2 = jnp.dot(h, w2_ref[:], preferred_element_type=jnp.float32) + b2_ref[:]
    out_ref[:] = _layernorm(x + v2, ln2w_ref[:], ln2b_ref[:])


def _row_block(i):
    return (i, 0)


def _pinned(i):
    return (0, 0)


_tc_fused = pl.pallas_call(
    _tc_body,
    grid=(_N // _ROWS,),
    in_specs=[
        pl.BlockSpec((_ROWS, _D), _row_block),
        pl.BlockSpec((_ROWS, 1), _row_block),
        pl.BlockSpec((_ROWS, 1), _row_block),
        pl.BlockSpec((_D, _D), _pinned),
        pl.BlockSpec((1, _D), _pinned),
        pl.BlockSpec((_D, _D), _pinned),
        pl.BlockSpec((1, _D), _pinned),
        pl.BlockSpec((1, _D), _pinned),
        pl.BlockSpec((1, _D), _pinned),
        pl.BlockSpec((_D, _D), _pinned),
        pl.BlockSpec((1, _D), _pinned),
        pl.BlockSpec((_D, _D), _pinned),
        pl.BlockSpec((1, _D), _pinned),
        pl.BlockSpec((1, _D), _pinned),
        pl.BlockSpec((1, _D), _pinned),
    ],
    out_specs=pl.BlockSpec((_ROWS, _D), _row_block),
    out_shape=jax.ShapeDtypeStruct((_N, _D), jnp.float32),
    compiler_params=pltpu.CompilerParams(
        dimension_semantics=("arbitrary",),
    ),
)


def kernel(q, k, v, self_indices, neighbor_indices, Wq, bq, Wk, bk, Wv, bv,
           Wo, bo, ln1_w, ln1_b, W1, b1, W2, b2, ln2_w, ln2_b):
    counts = _sc_count()(self_indices)
    c0 = counts[0:_N].reshape(_N, 1)
    c1 = counts[_NPAD:_NPAD + _N].reshape(_N, 1)
    row = lambda a: a.reshape(1, _D)
    return _tc_fused(
        v, c0, c1,
        Wv.T, row(bv), Wo.T, row(bo),
        row(ln1_w), row(ln1_b),
        W1.T, row(b1), W2.T, row(b2),
        row(ln2_w), row(ln2_b),
    )


# speedup vs baseline: 959.0455x; 1.1144x over previous
import functools

import jax
import jax.numpy as jnp
from jax import lax
from jax.experimental import pallas as pl
from jax.experimental.pallas import tpu as pltpu
from jax.experimental.pallas import tpu_sc as plsc

_N = 10000
_E = 320000
_D = 128

_NC = 2
_NS = 16
_NW = _NC * _NS
_EPW = _E // _NW
_CHUNK = 640
_NPAD = _NS * _CHUNK

_ROWS = 2048


def _sc_count_body(idx_hbm, out_hbm, idx_v, ones_v, zeros_v, shared):
    cid = lax.axis_index("c")
    sid = lax.axis_index("s")
    wid = sid * _NC + cid

    def fill_ones(i, _):
        ones_v[pl.ds(i * 16, 16)] = jnp.full((16,), 1.0, jnp.float32)
        return 0

    lax.fori_loop(0, _EPW // 16, fill_ones, 0)

    def fill_zeros(i, _):
        zeros_v[pl.ds(i * 16, 16)] = jnp.zeros((16,), jnp.float32)
        return 0

    lax.fori_loop(0, _CHUNK // 16, fill_zeros, 0)

    pltpu.sync_copy(zeros_v, shared.at[pl.ds(sid * _CHUNK, _CHUNK)])
    pltpu.sync_copy(idx_hbm.at[pl.ds(wid * _EPW, _EPW)], idx_v)
    plsc.subcore_barrier()
    pltpu.sync_copy(ones_v, shared.at[idx_v], add=True)
    plsc.subcore_barrier()
    pltpu.sync_copy(
        shared.at[pl.ds(sid * _CHUNK, _CHUNK)],
        out_hbm.at[pl.ds(cid * _NPAD + sid * _CHUNK, _CHUNK)],
    )


@functools.cache
def _sc_count():
    return pl.kernel(
        _sc_count_body,
        out_type=jax.ShapeDtypeStruct((_NC * _NPAD,), jnp.float32),
        mesh=plsc.VectorSubcoreMesh(core_axis_name="c", subcore_axis_name="s",
                                    num_cores=_NC, num_subcores=_NS),
        scratch_types=[
            pltpu.VMEM((_EPW,), jnp.int32),
            pltpu.VMEM((_EPW,), jnp.float32),
            pltpu.VMEM((_CHUNK,), jnp.float32),
            pltpu.VMEM_SHARED((_NPAD,), jnp.float32),
        ],
    )


def _layernorm(x, w, b):
    m = jnp.mean(x, axis=-1, keepdims=True)
    var = jnp.mean((x - m) * (x - m), axis=-1, keepdims=True)
    return (x - m) * lax.rsqrt(var + 1e-5) * w + b


def _dot_nt(x, w):
    return lax.dot_general(x, w, (((1,), (1,)), ((), ())),
                           preferred_element_type=jnp.float32)


def _tc_body(v_ref, c0_ref, c1_ref, wv_ref, bv_ref, wo_ref, bo_ref,
             ln1w_ref, ln1b_ref, w1_ref, b1_ref, w2_ref, b2_ref,
             ln2w_ref, ln2b_ref, out_ref):
    xv = v_ref[:]
    ind = jnp.where(c0_ref[:] + c1_ref[:] > 0.0, 1.0, 0.0)
    vl = _dot_nt(xv, wv_ref[:]) + bv_ref[:]
    agg = vl * ind
    v2 = _dot_nt(agg, wo_ref[:]) + bo_ref[:]
    x = _layernorm(xv + v2, ln1w_ref[:], ln1b_ref[:])
    h = jnp.maximum(_dot_nt(x, w1_ref[:]) + b1_ref[:], 0.0)
    v2 = _dot_nt(h, w2_ref[:]) + b2_ref[:]
    out_ref[:] = _layernorm(x + v2, ln2w_ref[:], ln2b_ref[:])


def _row_block(i):
    return (i, 0)


def _pinned(i):
    return (0, 0)


_GRID = _NPAD // _ROWS


def _c1_block(i):
    return (i + _GRID, 0)


_tc_fused = pl.pallas_call(
    _tc_body,
    grid=(_GRID,),
    in_specs=[
        pl.BlockSpec((_ROWS, _D), _row_block),
        pl.BlockSpec((_ROWS, 1), _row_block),
        pl.BlockSpec((_ROWS, 1), _c1_block),
        pl.BlockSpec((_D, _D), _pinned),
        pl.BlockSpec((1, _D), _pinned),
        pl.BlockSpec((_D, _D), _pinned),
        pl.BlockSpec((1, _D), _pinned),
        pl.BlockSpec((1, _D), _pinned),
        pl.BlockSpec((1, _D), _pinned),
        pl.BlockSpec((_D, _D), _pinned),
        pl.BlockSpec((1, _D), _pinned),
        pl.BlockSpec((_D, _D), _pinned),
        pl.BlockSpec((1, _D), _pinned),
        pl.BlockSpec((1, _D), _pinned),
        pl.BlockSpec((1, _D), _pinned),
    ],
    out_specs=pl.BlockSpec((_ROWS, _D), _row_block),
    out_shape=jax.ShapeDtypeStruct((_N, _D), jnp.float32),
    compiler_params=pltpu.CompilerParams(
        dimension_semantics=("arbitrary",),
    ),
)


def kernel(q, k, v, self_indices, neighbor_indices, Wq, bq, Wk, bk, Wv, bv,
           Wo, bo, ln1_w, ln1_b, W1, b1, W2, b2, ln2_w, ln2_b):
    counts = _sc_count()(self_indices).reshape(_NC * _NPAD, 1)
    row = lambda a: a.reshape(1, _D)
    return _tc_fused(
        v, counts, counts,
        Wv, row(bv), Wo, row(bo),
        row(ln1_w), row(ln1_b),
        W1, row(b1), W2, row(b2),
        row(ln2_w), row(ln2_b),
    )


# speedup vs baseline: 1121.7493x; 1.1697x over previous
import functools

import jax
import jax.numpy as jnp
from jax import lax
from jax.experimental import pallas as pl
from jax.experimental.pallas import tpu as pltpu
from jax.experimental.pallas import tpu_sc as plsc

_N = 10000
_E = 320000
_D = 128

_NC = 2
_NS = 16
_NW = _NC * _NS
_EPW = _E // _NW
_CHUNK = 640
_NPAD = _NS * _CHUNK

_ROWS = 2048


def _sc_count_body(idx_hbm, ones_hbm, out_hbm, idx_v, ones_v, shared):
    cid = lax.axis_index("c")
    sid = lax.axis_index("s")
    wid = sid * _NC + cid

    pltpu.sync_copy(ones_hbm, ones_v)
    pltpu.sync_copy(idx_hbm.at[pl.ds(wid * _EPW, _EPW)], idx_v)
    pltpu.sync_copy(ones_v.at[pl.ds(_EPW, _CHUNK)],
                    shared.at[pl.ds(sid * _CHUNK, _CHUNK)])
    plsc.subcore_barrier()
    pltpu.sync_copy(ones_v.at[pl.ds(0, _EPW)], shared.at[idx_v], add=True)
    plsc.subcore_barrier()
    pltpu.sync_copy(
        shared.at[pl.ds(sid * _CHUNK, _CHUNK)],
        out_hbm.at[pl.ds(cid * _NPAD + sid * _CHUNK, _CHUNK)],
    )


@functools.cache
def _sc_count():
    return pl.kernel(
        _sc_count_body,
        out_type=jax.ShapeDtypeStruct((_NC * _NPAD,), jnp.float32),
        mesh=plsc.VectorSubcoreMesh(core_axis_name="c", subcore_axis_name="s",
                                    num_cores=_NC, num_subcores=_NS),
        scratch_types=[
            pltpu.VMEM((_EPW,), jnp.int32),
            pltpu.VMEM((_EPW + _CHUNK,), jnp.float32),
            pltpu.VMEM_SHARED((_NPAD,), jnp.float32),
        ],
    )


def _layernorm(x, w, b):
    m = jnp.mean(x, axis=-1, keepdims=True)
    var = jnp.mean((x - m) * (x - m), axis=-1, keepdims=True)
    return (x - m) * lax.rsqrt(var + 1e-5) * w + b


def _dot_nt(x, w):
    return lax.dot_general(x, w, (((1,), (1,)), ((), ())),
                           preferred_element_type=jnp.float32)


def _tc_body(v_ref, c0_ref, c1_ref, wv_ref, bv_ref, wo_ref, bo_ref,
             ln1w_ref, ln1b_ref, w1_ref, b1_ref, w2_ref, b2_ref,
             ln2w_ref, ln2b_ref, out_ref):
    xv = v_ref[:]
    ct = (c0_ref[:] + c1_ref[:]).T
    cnt = jnp.concatenate(
        [lax.slice(ct, (0, a), (_D, a + 1)) for a in range(_ROWS // _D)],
        axis=0)
    ind = jnp.where(cnt > 0.0, 1.0, 0.0)
    vl = _dot_nt(xv, wv_ref[:]) + bv_ref[:]
    agg = vl * ind
    v2 = _dot_nt(agg, wo_ref[:]) + bo_ref[:]
    x = _layernorm(xv + v2, ln1w_ref[:], ln1b_ref[:])
    h = jnp.maximum(_dot_nt(x, w1_ref[:]) + b1_ref[:], 0.0)
    v2 = _dot_nt(h, w2_ref[:]) + b2_ref[:]
    out_ref[:] = _layernorm(x + v2, ln2w_ref[:], ln2b_ref[:])


def _row_block(i):
    return (i, 0)


def _pinned(i):
    return (0, 0)


_GRID = _NPAD // _ROWS


def _c1_block(i):
    return (i + _GRID, 0)


_tc_fused = pl.pallas_call(
    _tc_body,
    grid=(_GRID,),
    in_specs=[
        pl.BlockSpec((_ROWS, _D), _row_block),
        pl.BlockSpec((_ROWS // _D, _D), _row_block),
        pl.BlockSpec((_ROWS // _D, _D), _c1_block),
        pl.BlockSpec((_D, _D), _pinned),
        pl.BlockSpec((1, _D), _pinned),
        pl.BlockSpec((_D, _D), _pinned),
        pl.BlockSpec((1, _D), _pinned),
        pl.BlockSpec((1, _D), _pinned),
        pl.BlockSpec((1, _D), _pinned),
        pl.BlockSpec((_D, _D), _pinned),
        pl.BlockSpec((1, _D), _pinned),
        pl.BlockSpec((_D, _D), _pinned),
        pl.BlockSpec((1, _D), _pinned),
        pl.BlockSpec((1, _D), _pinned),
        pl.BlockSpec((1, _D), _pinned),
    ],
    out_specs=pl.BlockSpec((_ROWS, _D), _row_block),
    out_shape=jax.ShapeDtypeStruct((_N, _D), jnp.float32),
    compiler_params=pltpu.CompilerParams(
        dimension_semantics=("arbitrary",),
    ),
)


def kernel(q, k, v, self_indices, neighbor_indices, Wq, bq, Wk, bk, Wv, bv,
           Wo, bo, ln1_w, ln1_b, W1, b1, W2, b2, ln2_w, ln2_b):
    payload = jnp.concatenate([jnp.ones((_EPW,), jnp.float32),
                               jnp.zeros((_CHUNK,), jnp.float32)])
    counts = _sc_count()(self_indices, payload)
    counts = counts.reshape(_NC * _NPAD // _D, _D)
    row = lambda a: a.reshape(1, _D)
    return _tc_fused(
        v, counts, counts,
        Wv, row(bv), Wo, row(bo),
        row(ln1_w), row(ln1_b),
        W1, row(b1), W2, row(b2),
        row(ln2_w), row(ln2_b),
    )
